# baseline (device time: 17363 ns/iter reference)
import functools

import jax
import jax.numpy as jnp
from jax import lax
from jax.experimental import pallas as pl
from jax.experimental.pallas import tpu as pltpu

N_DEV = 16
GROUP = 4
HALVES = 2
EPS = 1e-5


def kernel(x, t_emb, W_scale, W_shift):
    b, s, c_shard = x.shape
    c_global = c_shard * N_DEV
    s2 = s // HALVES

    def body(x_ref, t_ref, ws_ref, wsh_ref, out_ref, comm_ref, send_sems, recv_sems):
        my = lax.axis_index("i")
        plane = (my // GROUP) * GROUP
        p = lax.rem(my, GROUP)

        plane_peers = [plane + lax.rem(p + j, GROUP) for j in range(1, GROUP)]
        col_peers = [lax.rem(my + GROUP * k, N_DEV) for k in range(1, GROUP)]
        peers = plane_peers + col_peers

        barrier = pltpu.get_barrier_semaphore()
        for peer in peers:
            pl.semaphore_signal(
                barrier, inc=1,
                device_id=(peer,), device_id_type=pl.DeviceIdType.MESH,
            )

        def sem_ix(phase, h, j):
            return (GROUP - 1) * (HALVES * phase + h) + (j - 1)

        def p1_rdma(h, j):
            return pltpu.make_async_remote_copy(
                src_ref=comm_ref.at[0, :, pl.ds(h * s2, s2)],
                dst_ref=comm_ref.at[j, :, pl.ds(h * s2, s2)],
                send_sem=send_sems.at[sem_ix(0, h, j)],
                recv_sem=recv_sems.at[sem_ix(0, h, j)],
                device_id=(plane_peers[j - 1],),
                device_id_type=pl.DeviceIdType.MESH,
            )

        def p2_rdma(h, k):
            return pltpu.make_async_remote_copy(
                src_ref=comm_ref.at[GROUP, :, pl.ds(h * s2, s2)],
                dst_ref=comm_ref.at[GROUP + k, :, pl.ds(h * s2, s2)],
                send_sem=send_sems.at[sem_ix(1, h, k)],
                recv_sem=recv_sems.at[sem_ix(1, h, k)],
                device_id=(col_peers[k - 1],),
                device_id_type=pl.DeviceIdType.MESH,
            )

        xv = x_ref[...]
        stats_h = []
        for h in range(HALVES):
            xh = xv[:, h * s2 : (h + 1) * s2, :]
            st = jnp.concatenate(
                [jnp.sum(xh, axis=-1), jnp.sum(xh * xh, axis=-1)], axis=0
            )
            stats_h.append(st)
            comm_ref[0, :, h * s2 : (h + 1) * s2] = st.astype(jnp.bfloat16)

        pl.semaphore_wait(barrier, len(peers))

        p1 = {h: [p1_rdma(h, j) for j in range(1, GROUP)] for h in range(HALVES)}
        for h in range(HALVES):
            for rdma in p1[h]:
                rdma.start()

        scale = jnp.dot(t_ref[...], ws_ref[...], preferred_element_type=jnp.float32)
        shift = jnp.dot(t_ref[...], wsh_ref[...], preferred_element_type=jnp.float32)

        p2 = {}
        plane_tot = {}
        for h in range(HALVES):
            for rdma in p1[h]:
                rdma.wait_recv()
            plane_tot[h] = jnp.sum(
                comm_ref[0:GROUP, :, h * s2 : (h + 1) * s2].astype(jnp.float32),
                axis=0,
            )
            comm_ref[GROUP, :, h * s2 : (h + 1) * s2] = plane_tot[h].astype(
                jnp.bfloat16
            )
            p2[h] = [p2_rdma(h, k) for k in range(1, GROUP)]
            for rdma in p2[h]:
                rdma.start()

        xb = xv.astype(jnp.bfloat16)
        sc = (1.0 + scale).astype(jnp.bfloat16)[:, None, :]
        sh = shift.astype(jnp.bfloat16)[:, None, :]

        tot_h = []
        for h in range(HALVES):
            for rdma in p2[h]:
                rdma.wait_recv()
            tot_h.append(
                plane_tot[h]
                + jnp.sum(
                    comm_ref[GROUP + 1 : 2 * GROUP, :, h * s2 : (h + 1) * s2].astype(
                        jnp.float32
                    ),
                    axis=0,
                )
            )
        total = jnp.concatenate(tot_h, axis=1)

        @functools.partial(pl.run_scoped, sem=pltpu.SemaphoreType.REGULAR)
        def _(sem):
            for peer in peers:
                pl.semaphore_signal(
                    sem, inc=1,
                    device_id=(peer,), device_id_type=pl.DeviceIdType.MESH,
                )

            mean = total[0:b, :] / c_global
            meansq = total[b : 2 * b, :] / c_global
            var = meansq - mean * mean
            inv = lax.rsqrt(var + EPS)
            mb = (mean * inv).astype(jnp.bfloat16)[:, :, None]
            ib = inv.astype(jnp.bfloat16)[:, :, None]
            h_ = xb * ib - mb
            out_ref[...] = (h_ * sc + sh).astype(out_ref.dtype)

            for h in range(HALVES):
                for rdma in p1[h] + p2[h]:
                    rdma.wait_send()
            pl.semaphore_wait(sem, len(peers))

    n_sems = 2 * HALVES * (GROUP - 1)
    return pl.pallas_call(
        body,
        out_shape=jax.ShapeDtypeStruct((b, s, c_shard), jnp.bfloat16),
        in_specs=[pl.BlockSpec(memory_space=pltpu.VMEM)] * 4,
        out_specs=pl.BlockSpec(memory_space=pltpu.VMEM),
        scratch_shapes=[
            pltpu.VMEM((2 * GROUP, 2 * b, s), jnp.bfloat16),
            pltpu.SemaphoreType.DMA((n_sems,)),
            pltpu.SemaphoreType.DMA((n_sems,)),
        ],
        compiler_params=pltpu.CompilerParams(collective_id=0),
    )(x, t_emb, W_scale, W_shift)
